# baseline (speedup 1.0000x reference)
import functools

import jax
import jax.numpy as jnp
from jax import lax
from jax.experimental import pallas as pl
from jax.experimental.pallas import tpu as pltpu
from jax.experimental.pallas import tpu_sc as plsc

_ROWS = 16384
_COLS = 200

_RTC = 0
_BR = 2048
_RSC = _ROWS - _RTC
_NW = 32
_RPW = _RSC // _NW
_CH = 64
_NCH = _RPW // _CH

_DCOLS = [16 * v for v in range(12)] + [184]


def _tc_body(p_ref, t_ref, m_ref, out_ref, acc_ref):
    i = pl.program_id(0)

    @pl.when(i == 0)
    def _init():
        acc_ref[0] = 0.0
        acc_ref[1] = 0.0

    p = p_ref[...]
    t = t_ref[...]
    m = m_ref[...]
    elem = jnp.exp(p) - t * p
    acc_ref[0] += jnp.sum(jnp.where(m, elem, 0.0))
    acc_ref[1] += jnp.sum(m.astype(jnp.float32))

    @pl.when(i == pl.num_programs(0) - 1)
    def _fin():
        out_ref[0, 0] = acc_ref[0]
        out_ref[0, 1] = acc_ref[1]


def _sc_body(p_hbm, t_hbm, m_hbm, sum_out, cnt_out, pbuf, tbuf, mbuf, accv, cntv):
    wid = lax.axis_index("s") * 2 + lax.axis_index("c")
    row0 = wid * _RPW

    acc = jnp.zeros((16,), jnp.float32)
    cnt = jnp.zeros((16,), jnp.float32)

    for c in range(_NCH):
        r0 = row0 + c * _CH
        pltpu.sync_copy(p_hbm.at[pl.ds(r0, _CH), :], pbuf)
        pltpu.sync_copy(t_hbm.at[pl.ds(r0, _CH), :], tbuf)
        pltpu.sync_copy(m_hbm.at[pl.ds(r0, _CH), :], mbuf)

        def _row(r, carry):
            acc, cnt = carry
            iota = lax.iota(jnp.int32, 16)
            tailw = (1 - lax.shift_right_logical(iota - 8, 31)).astype(
                jnp.float32
            )
            for d in _DCOLS:
                mf = mbuf[r, pl.ds(d, 16)]
                if d == 184:
                    mf = mf * tailw
                p = pbuf[r, pl.ds(d, 16)]
                t = tbuf[r, pl.ds(d, 16)]
                acc = acc + mf * (jnp.exp(p) - t * p)
                cnt = cnt + mf
            return acc, cnt

        acc, cnt = lax.fori_loop(0, _CH, _row, (acc, cnt))

    accv[...] = acc
    cntv[...] = cnt
    pltpu.sync_copy(accv, sum_out.at[wid])
    pltpu.sync_copy(cntv, cnt_out.at[wid])


@jax.jit
def kernel(y_pred, y_true, mask):
    m_sc = mask.astype(jnp.float32)
    mesh = plsc.VectorSubcoreMesh(core_axis_name="c", subcore_axis_name="s")
    sc_run = functools.partial(
        pl.kernel,
        out_type=(
            jax.ShapeDtypeStruct((_NW, 16), jnp.float32),
            jax.ShapeDtypeStruct((_NW, 16), jnp.float32),
        ),
        mesh=mesh,
        scratch_types=[
            pltpu.VMEM((_CH, _COLS), jnp.float32),
            pltpu.VMEM((_CH, _COLS), jnp.float32),
            pltpu.VMEM((_CH, _COLS), jnp.float32),
            pltpu.VMEM((16,), jnp.float32),
            pltpu.VMEM((16,), jnp.float32),
        ],
    )(_sc_body)
    sums, cnts = sc_run(y_pred, y_true, m_sc)
    return jnp.sum(sums) / jnp.sum(cnts)


# speedup vs baseline: 1.0597x; 1.0597x over previous
import functools

import jax
import jax.numpy as jnp
from jax import lax
from jax.experimental import pallas as pl
from jax.experimental.pallas import tpu as pltpu
from jax.experimental.pallas import tpu_sc as plsc

_ROWS = 16384
_COLS = 200

_RTC = 14336
_BR = 2048
_RSC = _ROWS - _RTC
_NW = 32
_RPW = _RSC // _NW
_CH = 64
_NCH = _RPW // _CH

_DCOLS = [16 * v for v in range(12)] + [184]


def _tc_body(p_ref, t_ref, m_ref, out_ref, acc_ref):
    i = pl.program_id(0)

    @pl.when(i == 0)
    def _init():
        acc_ref[0] = 0.0
        acc_ref[1] = 0.0

    p = p_ref[...]
    t = t_ref[...]
    m = m_ref[...]
    elem = jnp.exp(p) - t * p
    acc_ref[0] += jnp.sum(jnp.where(m, elem, 0.0))
    acc_ref[1] += jnp.sum(m.astype(jnp.float32))

    @pl.when(i == pl.num_programs(0) - 1)
    def _fin():
        out_ref[0, 0] = acc_ref[0]
        out_ref[0, 1] = acc_ref[1]


def _sc_body(p_hbm, t_hbm, m_hbm, sum_out, cnt_out, pbuf, tbuf, mbuf, accv, cntv):
    wid = lax.axis_index("s") * 2 + lax.axis_index("c")
    row0 = _RTC + wid * _RPW

    acc = jnp.zeros((16,), jnp.float32)
    cnt = jnp.zeros((16,), jnp.float32)

    for c in range(_NCH):
        r0 = row0 + c * _CH
        pltpu.sync_copy(p_hbm.at[pl.ds(r0, _CH), :], pbuf)
        pltpu.sync_copy(t_hbm.at[pl.ds(r0, _CH), :], tbuf)
        pltpu.sync_copy(m_hbm.at[pl.ds(r0, _CH), :], mbuf)

        def _row(r, carry):
            acc, cnt = carry
            iota = lax.iota(jnp.int32, 16)
            tailw = (1 - lax.shift_right_logical(iota - 8, 31)).astype(
                jnp.float32
            )
            for d in _DCOLS:
                mf = mbuf[r, pl.ds(d, 16)]
                if d == 184:
                    mf = mf * tailw
                p = pbuf[r, pl.ds(d, 16)]
                t = tbuf[r, pl.ds(d, 16)]
                acc = acc + mf * (jnp.exp(p) - t * p)
                cnt = cnt + mf
            return acc, cnt

        acc, cnt = lax.fori_loop(0, _CH, _row, (acc, cnt))

    accv[...] = acc
    cntv[...] = cnt
    pltpu.sync_copy(accv, sum_out.at[wid])
    pltpu.sync_copy(cntv, cnt_out.at[wid])


@jax.jit
def kernel(y_pred, y_true, mask):
    m_sc = mask.astype(jnp.float32)
    mesh = plsc.VectorSubcoreMesh(core_axis_name="c", subcore_axis_name="s")
    sc_run = functools.partial(
        pl.kernel,
        out_type=(
            jax.ShapeDtypeStruct((_NW, 16), jnp.float32),
            jax.ShapeDtypeStruct((_NW, 16), jnp.float32),
        ),
        mesh=mesh,
        scratch_types=[
            pltpu.VMEM((_CH, _COLS), jnp.float32),
            pltpu.VMEM((_CH, _COLS), jnp.float32),
            pltpu.VMEM((_CH, _COLS), jnp.float32),
            pltpu.VMEM((16,), jnp.float32),
            pltpu.VMEM((16,), jnp.float32),
        ],
    )(_sc_body)
    sums, cnts = sc_run(y_pred, y_true, m_sc)

    out_tc = pl.pallas_call(
        _tc_body,
        grid=(_RTC // _BR,),
        in_specs=[
            pl.BlockSpec((_BR, _COLS), lambda i: (i, 0)),
            pl.BlockSpec((_BR, _COLS), lambda i: (i, 0)),
            pl.BlockSpec((_BR, _COLS), lambda i: (i, 0)),
        ],
        out_specs=pl.BlockSpec(memory_space=pltpu.SMEM),
        out_shape=jax.ShapeDtypeStruct((1, 2), jnp.float32),
        scratch_shapes=[pltpu.SMEM((2,), jnp.float32)],
    )(y_pred, y_true, mask)

    total = out_tc[0, 0] + jnp.sum(sums)
    count = out_tc[0, 1] + jnp.sum(cnts)
    return total / count


# speedup vs baseline: 1.1848x; 1.1180x over previous
import functools

import jax
import jax.numpy as jnp
from jax import lax
from jax.experimental import pallas as pl
from jax.experimental.pallas import tpu as pltpu
from jax.experimental.pallas import tpu_sc as plsc

_ROWS = 16384
_COLS = 200

_RTC = 14336
_BR = 2048
_RSC = _ROWS - _RTC
_NW = 32
_RPW = _RSC // _NW
_CH = 64
_NCH = _RPW // _CH

_DCOLS = [16 * v for v in range(12)] + [184]


def _tc_body(p_ref, t_ref, m_ref, out_ref, acc_ref):
    i = pl.program_id(0)

    @pl.when(i == 0)
    def _init():
        acc_ref[0] = 0.0
        acc_ref[1] = 0.0

    p = p_ref[...]
    t = t_ref[...]
    m = m_ref[...]
    elem = jnp.exp(p) - t * p
    acc_ref[0] += jnp.sum(jnp.where(m, elem, 0.0))
    acc_ref[1] += jnp.sum(m.astype(jnp.float32))

    @pl.when(i == pl.num_programs(0) - 1)
    def _fin():
        out_ref[0, 0] = acc_ref[0]
        out_ref[0, 1] = acc_ref[1]


def _sc_body(p_hbm, t_hbm, m_hbm, sum_out, cnt_out, pbuf, tbuf, mbuf, accv, cntv):
    wid = lax.axis_index("s") * 2 + lax.axis_index("c")
    row0 = wid * _RPW

    acc = jnp.zeros((16,), jnp.float32)
    cnt = jnp.zeros((16,), jnp.float32)

    for c in range(_NCH):
        r0 = row0 + c * _CH
        pltpu.sync_copy(p_hbm.at[pl.ds(r0, _CH), :], pbuf)
        pltpu.sync_copy(t_hbm.at[pl.ds(r0, _CH), :], tbuf)
        pltpu.sync_copy(m_hbm.at[pl.ds(r0, _CH), :], mbuf)

        def _row(r, carry):
            acc, cnt = carry
            iota = lax.iota(jnp.int32, 16)
            tailw = (1 - lax.shift_right_logical(iota - 8, 31)).astype(
                jnp.float32
            )
            for d in _DCOLS:
                mf = mbuf[r, pl.ds(d, 16)]
                if d == 184:
                    mf = mf * tailw
                p = pbuf[r, pl.ds(d, 16)]
                t = tbuf[r, pl.ds(d, 16)]
                acc = acc + mf * (jnp.exp(p) - t * p)
                cnt = cnt + mf
            return acc, cnt

        acc, cnt = lax.fori_loop(0, _CH, _row, (acc, cnt))

    accv[...] = acc
    cntv[...] = cnt
    pltpu.sync_copy(accv, sum_out.at[wid])
    pltpu.sync_copy(cntv, cnt_out.at[wid])


@jax.jit
def kernel(y_pred, y_true, mask):
    p_sc = lax.slice(y_pred, (_RTC, 0), (_ROWS, _COLS))
    t_sc = lax.slice(y_true, (_RTC, 0), (_ROWS, _COLS))
    m_sc = lax.slice(mask, (_RTC, 0), (_ROWS, _COLS)).astype(jnp.float32)
    mesh = plsc.VectorSubcoreMesh(core_axis_name="c", subcore_axis_name="s")
    sc_run = functools.partial(
        pl.kernel,
        out_type=(
            jax.ShapeDtypeStruct((_NW, 16), jnp.float32),
            jax.ShapeDtypeStruct((_NW, 16), jnp.float32),
        ),
        mesh=mesh,
        scratch_types=[
            pltpu.VMEM((_CH, _COLS), jnp.float32),
            pltpu.VMEM((_CH, _COLS), jnp.float32),
            pltpu.VMEM((_CH, _COLS), jnp.float32),
            pltpu.VMEM((16,), jnp.float32),
            pltpu.VMEM((16,), jnp.float32),
        ],
    )(_sc_body)
    sums, cnts = sc_run(p_sc, t_sc, m_sc)

    out_tc = pl.pallas_call(
        _tc_body,
        grid=(_RTC // _BR,),
        in_specs=[
            pl.BlockSpec((_BR, _COLS), lambda i: (i, 0)),
            pl.BlockSpec((_BR, _COLS), lambda i: (i, 0)),
            pl.BlockSpec((_BR, _COLS), lambda i: (i, 0)),
        ],
        out_specs=pl.BlockSpec(memory_space=pltpu.SMEM),
        out_shape=jax.ShapeDtypeStruct((1, 2), jnp.float32),
        scratch_shapes=[pltpu.SMEM((2,), jnp.float32)],
    )(y_pred, y_true, mask)

    total = out_tc[0, 0] + jnp.sum(sums)
    count = out_tc[0, 1] + jnp.sum(cnts)
    return total / count
